# Initial kernel scaffold; baseline (speedup 1.0000x reference)
import jax
import jax.numpy as jnp
from jax.experimental import pallas as pl


def kernel(x, T_atom, T_charge, T_chiral, T_arom, T_ring):
    raise NotImplementedError("write your pallas kernel here")


# speedup vs baseline: 1.4906x; 1.4906x over previous
import functools

import jax
import jax.numpy as jnp
from jax import lax
from jax.experimental import pallas as pl
from jax.experimental.pallas import tpu as pltpu
from jax.experimental.pallas import tpu_sc as plsc

N = 100000
D = 128
NCOLS = 16
NTAB = 5
NPASS = NCOLS - NTAB
OUT_W = NTAB * D + NPASS
CHUNK = 128
NUM_CHUNKS = -(-N // CHUNK)
LAST_START = N - CHUNK
NW = 32
MAX_ITERS = -(-NUM_CHUNKS // NW)


def _sc_body(xf_hbm, t0, t1, t2, t3, t4, out_hbm,
             xv, i0, i1, i2, i3, i4, asm_v, sem):
    tables = (t0, t1, t2, t3, t4)
    idxs = (i0, i1, i2, i3, i4)
    w = lax.axis_index("s") * 2 + lax.axis_index("c")

    def chunk_body(t, carry):
        i = w + t * NW

        @pl.when(i < NUM_CHUNKS)
        def _():
            r0 = jnp.minimum(i * CHUNK, LAST_START)
            pltpu.sync_copy(xf_hbm.at[pl.ds(r0 * NCOLS, CHUNK * NCOLS)], xv)
            lane = lax.iota(jnp.int32, 16)
            for g in range(CHUNK // 16):
                rows = lane * NCOLS + (16 * NCOLS * g)
                for c in range(NTAB):
                    vals = plsc.load_gather(xv, [rows + c])
                    idxs[c][pl.ds(16 * g, 16)] = vals.astype(jnp.int32)
            for r in range(CHUNK):
                asm_v[r, pl.ds(OUT_W - 16, 16)] = xv[pl.ds(NCOLS * r, 16)]
            handles = []
            for c in range(NTAB):
                handles.append(pltpu.async_copy(
                    tables[c].at[idxs[c]],
                    asm_v.at[:, pl.ds(c * D, D)], sem))
            for h in handles:
                h.wait()
            pltpu.sync_copy(asm_v, out_hbm.at[pl.ds(r0, CHUNK)])

        return carry

    lax.fori_loop(0, MAX_ITERS, chunk_body, 0)


@jax.jit
def kernel(x, T_atom, T_charge, T_chiral, T_arom, T_ring):
    mesh = plsc.VectorSubcoreMesh(core_axis_name="c", subcore_axis_name="s")
    run = functools.partial(
        pl.kernel,
        mesh=mesh,
        compiler_params=pltpu.CompilerParams(needs_layout_passes=False),
        out_type=jax.ShapeDtypeStruct((N, OUT_W), jnp.float32),
        scratch_types=[
            pltpu.VMEM((CHUNK * NCOLS,), jnp.float32),
            pltpu.VMEM((CHUNK,), jnp.int32),
            pltpu.VMEM((CHUNK,), jnp.int32),
            pltpu.VMEM((CHUNK,), jnp.int32),
            pltpu.VMEM((CHUNK,), jnp.int32),
            pltpu.VMEM((CHUNK,), jnp.int32),
            pltpu.VMEM((CHUNK, OUT_W), jnp.float32),
            pltpu.SemaphoreType.DMA,
        ],
    )(_sc_body)
    return run(x.reshape(-1), T_atom, T_charge, T_chiral, T_arom, T_ring)


# speedup vs baseline: 1.7470x; 1.1720x over previous
import functools

import jax
import jax.numpy as jnp
from jax import lax
from jax.experimental import pallas as pl
from jax.experimental.pallas import tpu as pltpu
from jax.experimental.pallas import tpu_sc as plsc

N = 100000
D = 128
NCOLS = 16
NTAB = 5
NPASS = NCOLS - NTAB
OUT_W = NTAB * D + NPASS
CHUNK = 80
NUM_CHUNKS = N // CHUNK
NW = 32
MAX_ITERS = -(-NUM_CHUNKS // NW)


def _sc_body(xf_hbm, t0, t1, t2, t3, t4, out_hbm,
             xv0, xv1, i0, i1, i2, i3, i4, j0, j1, j2, j3, j4,
             asm0, asm1, semx0, semx1, semg, semo0, semo1):
    tables = (t0, t1, t2, t3, t4)
    xvs = (xv0, xv1)
    idxsets = ((i0, i1, i2, i3, i4), (j0, j1, j2, j3, j4))
    asms = (asm0, asm1)
    semxs = (semx0, semx1)
    semos = (semo0, semo1)
    w = lax.axis_index("s") * 2 + lax.axis_index("c")

    def xin_copy(i, p):
        return pltpu.make_async_copy(
            xf_hbm.at[pl.ds(i * (CHUNK * NCOLS), CHUNK * NCOLS)],
            xvs[p], semxs[p])

    def out_copy(i, p):
        return pltpu.make_async_copy(
            asms[p], out_hbm.at[pl.ds(i * CHUNK, CHUNK)], semos[p])

    xin_copy(w, 0).start()

    lane = lax.iota(jnp.int32, 16)

    def step(tt, b):
        i = w + (2 * tt + b) * NW

        @pl.when(i < NUM_CHUNKS)
        def _():
            xv = xvs[b]
            asm_v = asms[b]
            idxs = idxsets[b]
            xin_copy(i, b).wait()
            @pl.when(i + NW < NUM_CHUNKS)
            def _():
                xin_copy(i + NW, 1 - b).start()
            for g in range(CHUNK // 16):
                rows = lane * NCOLS + (16 * NCOLS * g)
                for c in range(NTAB):
                    vals = plsc.load_gather(xv, [rows + c])
                    idxs[c][pl.ds(16 * g, 16)] = vals.astype(jnp.int32)
            @pl.when(tt >= 1)
            def _():
                out_copy(i, b).wait()
            for g in range(CHUNK // 16):
                rows = lane + 16 * g
                srcbase = lane * NCOLS + (16 * NCOLS * g)
                for k in range(NPASS):
                    vals = plsc.load_gather(xv, [srcbase + (NTAB + k)])
                    plsc.store_scatter(
                        asm_v, [rows, jnp.full((16,), NTAB * D + k, jnp.int32)],
                        vals)
            handles = []
            for c in range(NTAB):
                handles.append(pltpu.async_copy(
                    tables[c].at[idxs[c]],
                    asm_v.at[:, pl.ds(c * D, D)], semg))
            for h in handles:
                h.wait()
            out_copy(i, b).start()

    def loop_body(tt, carry):
        step(tt, 0)
        step(tt, 1)
        return carry

    lax.fori_loop(0, MAX_ITERS // 2, loop_body, 0)

    out_copy(w, 0).wait()
    out_copy(w, 1).wait()


@jax.jit
def kernel(x, T_atom, T_charge, T_chiral, T_arom, T_ring):
    mesh = plsc.VectorSubcoreMesh(core_axis_name="c", subcore_axis_name="s")
    run = functools.partial(
        pl.kernel,
        mesh=mesh,
        compiler_params=pltpu.CompilerParams(needs_layout_passes=False),
        out_type=jax.ShapeDtypeStruct((N, OUT_W), jnp.float32),
        scratch_types=[
            pltpu.VMEM((CHUNK * NCOLS,), jnp.float32),
            pltpu.VMEM((CHUNK * NCOLS,), jnp.float32),
            pltpu.VMEM((CHUNK,), jnp.int32),
            pltpu.VMEM((CHUNK,), jnp.int32),
            pltpu.VMEM((CHUNK,), jnp.int32),
            pltpu.VMEM((CHUNK,), jnp.int32),
            pltpu.VMEM((CHUNK,), jnp.int32),
            pltpu.VMEM((CHUNK,), jnp.int32),
            pltpu.VMEM((CHUNK,), jnp.int32),
            pltpu.VMEM((CHUNK,), jnp.int32),
            pltpu.VMEM((CHUNK,), jnp.int32),
            pltpu.VMEM((CHUNK,), jnp.int32),
            pltpu.VMEM((CHUNK, OUT_W), jnp.float32),
            pltpu.VMEM((CHUNK, OUT_W), jnp.float32),
            pltpu.SemaphoreType.DMA,
            pltpu.SemaphoreType.DMA,
            pltpu.SemaphoreType.DMA,
            pltpu.SemaphoreType.DMA,
            pltpu.SemaphoreType.DMA,
        ],
    )(_sc_body)
    return run(x.reshape(-1), T_atom, T_charge, T_chiral, T_arom, T_ring)
